# Initial kernel scaffold; baseline (speedup 1.0000x reference)
import jax
import jax.numpy as jnp
from jax.experimental import pallas as pl


def kernel(x, edge_index, W1, b1, W2, b2):
    raise NotImplementedError("write your pallas kernel here")


# speedup vs baseline: 23.0685x; 23.0685x over previous
import functools

import jax
import jax.numpy as jnp
from jax import lax
from jax.experimental import pallas as pl
from jax.experimental.pallas import tpu as pltpu
from jax.experimental.pallas import tpu_sc as plsc

N = 10000
E = 320000
NC, NS = 2, 16
NW = NC * NS
BLK = 128
EPT = 10240
EP = EPT * NW
NBLK = EPT // BLK
NP = 10240
ROWS_PT = NP // NS
TC_BLK = 1000


def _edge_agg(feat, srcp, dstp, F):
  mesh = plsc.VectorSubcoreMesh(core_axis_name="c", subcore_axis_name="s")

  @functools.partial(
      pl.kernel,
      out_type=jax.ShapeDtypeStruct((NC, NP, F), jnp.float32),
      mesh=mesh,
      scratch_types=[
          pltpu.VMEM((NBLK, BLK), jnp.int32),
          pltpu.VMEM((NBLK, BLK), jnp.int32),
          pltpu.VMEM((BLK, F), jnp.float32),
          pltpu.VMEM((ROWS_PT, F), jnp.float32),
          pltpu.VMEM_SHARED((NP, F), jnp.float32),
          pltpu.SemaphoreType.DMA,
      ],
      compiler_params=pltpu.CompilerParams(use_tc_tiling_on_sc=False),
  )
  def body(feat_hbm, srcp_hbm, dstp_hbm, out_hbm,
           src_v, dst_v, rows_v, zbuf_v, acc_sh, sem):
    cid = lax.axis_index("c")
    sid = lax.axis_index("s")
    wid = sid * NC + cid

    def zero_row(r, carry):
      for j in range(F // 16):
        zbuf_v[r, pl.ds(j * 16, 16)] = jnp.zeros((16,), jnp.float32)
      return carry
    lax.fori_loop(0, ROWS_PT, zero_row, 0)
    pltpu.sync_copy(zbuf_v, acc_sh.at[pl.ds(sid * ROWS_PT, ROWS_PT)])
    plsc.subcore_barrier()

    pltpu.sync_copy(srcp_hbm.at[pl.ds(wid * NBLK, NBLK)], src_v)
    pltpu.sync_copy(dstp_hbm.at[pl.ds(wid * NBLK, NBLK)], dst_v)

    def step(j, carry):
      pltpu.async_copy(feat_hbm.at[src_v.at[j]], rows_v, sem).wait()
      pltpu.sync_copy(rows_v, acc_sh.at[dst_v.at[j]], add=True)
      return carry
    lax.fori_loop(0, NBLK, step, 0)

    plsc.subcore_barrier()
    pltpu.sync_copy(acc_sh.at[pl.ds(sid * ROWS_PT, ROWS_PT)], zbuf_v)
    pltpu.sync_copy(zbuf_v, out_hbm.at[cid, pl.ds(sid * ROWS_PT, ROWS_PT)])

  return body(feat, srcp, dstp)


def _deg_hist(dstp):
  F = 16
  mesh = plsc.VectorSubcoreMesh(core_axis_name="c", subcore_axis_name="s")

  @functools.partial(
      pl.kernel,
      out_type=jax.ShapeDtypeStruct((NC, NP, F), jnp.float32),
      mesh=mesh,
      scratch_types=[
          pltpu.VMEM((NBLK, BLK), jnp.int32),
          pltpu.VMEM((BLK, F), jnp.float32),
          pltpu.VMEM((ROWS_PT, F), jnp.float32),
          pltpu.VMEM_SHARED((NP, F), jnp.float32),
      ],
      compiler_params=pltpu.CompilerParams(use_tc_tiling_on_sc=False),
  )
  def body(dstp_hbm, out_hbm, dst_v, ones_v, zbuf_v, acc_sh):
    cid = lax.axis_index("c")
    sid = lax.axis_index("s")
    wid = sid * NC + cid

    def zero_row(r, carry):
      zbuf_v[r, pl.ds(0, 16)] = jnp.zeros((16,), jnp.float32)
      return carry
    lax.fori_loop(0, ROWS_PT, zero_row, 0)
    pltpu.sync_copy(zbuf_v, acc_sh.at[pl.ds(sid * ROWS_PT, ROWS_PT)])

    def one_row(r, carry):
      ones_v[r, pl.ds(0, 16)] = jnp.ones((16,), jnp.float32)
      return carry
    lax.fori_loop(0, BLK, one_row, 0)
    plsc.subcore_barrier()

    pltpu.sync_copy(dstp_hbm.at[pl.ds(wid * NBLK, NBLK)], dst_v)

    def step(j, carry):
      pltpu.sync_copy(ones_v, acc_sh.at[dst_v.at[j]], add=True)
      return carry
    lax.fori_loop(0, NBLK, step, 0)

    plsc.subcore_barrier()
    pltpu.sync_copy(acc_sh.at[pl.ds(sid * ROWS_PT, ROWS_PT)], zbuf_v)
    pltpu.sync_copy(zbuf_v, out_hbm.at[cid, pl.ds(sid * ROWS_PT, ROWS_PT)])

  return body(dstp)


def _dinv_from(degp_blk):
  deg = degp_blk[0, :, 0:1] + degp_blk[1, :, 0:1] + 1.0
  return lax.rsqrt(deg)


def _tc1(x, W1, degp):
  def body(x_ref, w_ref, degp_ref, o_ref):
    dinv = _dinv_from(degp_ref)
    o_ref[...] = dinv * jnp.dot(x_ref[...], w_ref[...],
                                preferred_element_type=jnp.float32)

  grid = N // TC_BLK
  return pl.pallas_call(
      body,
      grid=(grid,),
      in_specs=[
          pl.BlockSpec((TC_BLK, 128), lambda i: (i, 0)),
          pl.BlockSpec((128, 32), lambda i: (0, 0)),
          pl.BlockSpec((NC, TC_BLK, 16), lambda i: (0, i, 0)),
      ],
      out_specs=pl.BlockSpec((TC_BLK, 32), lambda i: (i, 0)),
      out_shape=jax.ShapeDtypeStruct((N, 32), jnp.float32),
  )(x, W1, degp)


def _tc2(r1p, h1s, degp, b1, W2):
  def body(r_ref, h_ref, degp_ref, b_ref, w_ref, o_ref):
    dinv = _dinv_from(degp_ref)
    z = dinv * (r_ref[0] + r_ref[1] + h_ref[...]) + b_ref[...]
    z = jnp.maximum(z, 0.0)
    o_ref[...] = dinv * jnp.dot(z, w_ref[...],
                                preferred_element_type=jnp.float32)

  grid = N // TC_BLK
  return pl.pallas_call(
      body,
      grid=(grid,),
      in_specs=[
          pl.BlockSpec((NC, TC_BLK, 32), lambda i: (0, i, 0)),
          pl.BlockSpec((TC_BLK, 32), lambda i: (i, 0)),
          pl.BlockSpec((NC, TC_BLK, 16), lambda i: (0, i, 0)),
          pl.BlockSpec((1, 32), lambda i: (0, 0)),
          pl.BlockSpec((32, 16), lambda i: (0, 0)),
      ],
      out_specs=pl.BlockSpec((TC_BLK, 16), lambda i: (i, 0)),
      out_shape=jax.ShapeDtypeStruct((N, 16), jnp.float32),
  )(r1p, h1s, degp, b1, W2)


def _tc3(r2p, h2s, degp, b2):
  def body(r_ref, h_ref, degp_ref, b_ref, o_ref):
    dinv = _dinv_from(degp_ref)
    o_ref[...] = dinv * (r_ref[0] + r_ref[1] + h_ref[...]) + b_ref[...]

  grid = N // TC_BLK
  return pl.pallas_call(
      body,
      grid=(grid,),
      in_specs=[
          pl.BlockSpec((NC, TC_BLK, 16), lambda i: (0, i, 0)),
          pl.BlockSpec((TC_BLK, 16), lambda i: (i, 0)),
          pl.BlockSpec((NC, TC_BLK, 16), lambda i: (0, i, 0)),
          pl.BlockSpec((1, 16), lambda i: (0, 0)),
      ],
      out_specs=pl.BlockSpec((TC_BLK, 16), lambda i: (i, 0)),
      out_shape=jax.ShapeDtypeStruct((N, 16), jnp.float32),
  )(r2p, h2s, degp, b2)


@jax.jit
def kernel(x, edge_index, W1, b1, W2, b2):
  ei = edge_index.astype(jnp.int32)
  pad = EP - E
  srcp = jnp.concatenate([ei[0], jnp.zeros((pad,), jnp.int32)])
  dstp = jnp.concatenate([ei[1], jnp.full((pad,), N, jnp.int32)])
  srcp = srcp.reshape(EP // BLK, BLK)
  dstp = dstp.reshape(EP // BLK, BLK)

  degp = _deg_hist(dstp)
  h1s = _tc1(x, W1, degp)
  r1p = _edge_agg(h1s, srcp, dstp, 32)
  h2s = _tc2(r1p, h1s, degp, b1.reshape(1, 32), W2)
  r2p = _edge_agg(h2s, srcp, dstp, 16)
  return _tc3(r2p, h2s, degp, b2.reshape(1, 16))


# speedup vs baseline: 28.7470x; 1.2462x over previous
import functools

import jax
import jax.numpy as jnp
from jax import lax
from jax.experimental import pallas as pl
from jax.experimental.pallas import tpu as pltpu
from jax.experimental.pallas import tpu_sc as plsc

N = 10000
E = 320000
NC, NS = 2, 16
NW = NC * NS
BLK = 128
EPT = 10240
EP = EPT * NW
NBLK = EPT // BLK
NP = 10240
ROWS_PT = NP // NS
TC_BLK = 1000


def _edge_agg(feat, srcp, dstp, F):
  mesh = plsc.VectorSubcoreMesh(core_axis_name="c", subcore_axis_name="s")

  @functools.partial(
      pl.kernel,
      out_type=jax.ShapeDtypeStruct((NC, NP, F), jnp.float32),
      mesh=mesh,
      scratch_types=[
          pltpu.VMEM((NBLK, BLK), jnp.int32),
          pltpu.VMEM((NBLK, BLK), jnp.int32),
          pltpu.VMEM((BLK, F), jnp.float32),
          pltpu.VMEM((BLK, F), jnp.float32),
          pltpu.VMEM((ROWS_PT, F), jnp.float32),
          pltpu.VMEM_SHARED((NP, F), jnp.float32),
          pltpu.SemaphoreType.DMA,
          pltpu.SemaphoreType.DMA,
      ],
      compiler_params=pltpu.CompilerParams(use_tc_tiling_on_sc=False),
  )
  def body(feat_hbm, srcp_hbm, dstp_hbm, out_hbm,
           src_v, dst_v, rows0_v, rows1_v, zbuf_v, acc_sh, sem0, sem1):
    cid = lax.axis_index("c")
    sid = lax.axis_index("s")
    wid = sid * NC + cid

    def zero_row(r, carry):
      for j in range(F // 16):
        zbuf_v[r, pl.ds(j * 16, 16)] = jnp.zeros((16,), jnp.float32)
      return carry
    lax.fori_loop(0, ROWS_PT, zero_row, 0)
    pltpu.sync_copy(zbuf_v, acc_sh.at[pl.ds(sid * ROWS_PT, ROWS_PT)])
    plsc.subcore_barrier()

    pltpu.sync_copy(srcp_hbm.at[pl.ds(wid * NBLK, NBLK)], src_v)
    pltpu.sync_copy(dstp_hbm.at[pl.ds(wid * NBLK, NBLK)], dst_v)

    pltpu.make_async_copy(feat_hbm.at[src_v.at[0]], rows0_v, sem0).start()

    def step(j2, carry):
      j = 2 * j2
      g1 = pltpu.make_async_copy(feat_hbm.at[src_v.at[j + 1]], rows1_v, sem1)
      g1.start()
      pltpu.make_async_copy(feat_hbm.at[src_v.at[j]], rows0_v, sem0).wait()
      pltpu.sync_copy(rows0_v, acc_sh.at[dst_v.at[j]], add=True)

      @pl.when(j2 + 1 < NBLK // 2)
      def _():
        pltpu.make_async_copy(
            feat_hbm.at[src_v.at[j + 2]], rows0_v, sem0).start()

      g1.wait()
      pltpu.sync_copy(rows1_v, acc_sh.at[dst_v.at[j + 1]], add=True)
      return carry
    lax.fori_loop(0, NBLK // 2, step, 0)

    plsc.subcore_barrier()
    pltpu.sync_copy(acc_sh.at[pl.ds(sid * ROWS_PT, ROWS_PT)], zbuf_v)
    pltpu.sync_copy(zbuf_v, out_hbm.at[cid, pl.ds(sid * ROWS_PT, ROWS_PT)])

  return body(feat, srcp, dstp)


def _deg_hist(dstp):
  F = 16
  mesh = plsc.VectorSubcoreMesh(core_axis_name="c", subcore_axis_name="s")

  @functools.partial(
      pl.kernel,
      out_type=jax.ShapeDtypeStruct((NC, NP, F), jnp.float32),
      mesh=mesh,
      scratch_types=[
          pltpu.VMEM((NBLK, BLK), jnp.int32),
          pltpu.VMEM((BLK, F), jnp.float32),
          pltpu.VMEM((ROWS_PT, F), jnp.float32),
          pltpu.VMEM_SHARED((NP, F), jnp.float32),
      ],
      compiler_params=pltpu.CompilerParams(use_tc_tiling_on_sc=False),
  )
  def body(dstp_hbm, out_hbm, dst_v, ones_v, zbuf_v, acc_sh):
    cid = lax.axis_index("c")
    sid = lax.axis_index("s")
    wid = sid * NC + cid

    def zero_row(r, carry):
      zbuf_v[r, pl.ds(0, 16)] = jnp.zeros((16,), jnp.float32)
      return carry
    lax.fori_loop(0, ROWS_PT, zero_row, 0)
    pltpu.sync_copy(zbuf_v, acc_sh.at[pl.ds(sid * ROWS_PT, ROWS_PT)])

    def one_row(r, carry):
      ones_v[r, pl.ds(0, 16)] = jnp.ones((16,), jnp.float32)
      return carry
    lax.fori_loop(0, BLK, one_row, 0)
    plsc.subcore_barrier()

    pltpu.sync_copy(dstp_hbm.at[pl.ds(wid * NBLK, NBLK)], dst_v)

    def step(j, carry):
      pltpu.sync_copy(ones_v, acc_sh.at[dst_v.at[j]], add=True)
      return carry
    lax.fori_loop(0, NBLK, step, 0)

    plsc.subcore_barrier()
    pltpu.sync_copy(acc_sh.at[pl.ds(sid * ROWS_PT, ROWS_PT)], zbuf_v)
    pltpu.sync_copy(zbuf_v, out_hbm.at[cid, pl.ds(sid * ROWS_PT, ROWS_PT)])

  return body(dstp)


def _dinv_from(degp_blk):
  deg = degp_blk[0, :, 0:1] + degp_blk[1, :, 0:1] + 1.0
  return lax.rsqrt(deg)


def _tc1(x, W1, degp):
  def body(x_ref, w_ref, degp_ref, o_ref):
    dinv = _dinv_from(degp_ref)
    o_ref[...] = dinv * jnp.dot(x_ref[...], w_ref[...],
                                preferred_element_type=jnp.float32)

  grid = N // TC_BLK
  return pl.pallas_call(
      body,
      grid=(grid,),
      in_specs=[
          pl.BlockSpec((TC_BLK, 128), lambda i: (i, 0)),
          pl.BlockSpec((128, 32), lambda i: (0, 0)),
          pl.BlockSpec((NC, TC_BLK, 16), lambda i: (0, i, 0)),
      ],
      out_specs=pl.BlockSpec((TC_BLK, 32), lambda i: (i, 0)),
      out_shape=jax.ShapeDtypeStruct((N, 32), jnp.float32),
  )(x, W1, degp)


def _tc2(r1p, h1s, degp, b1, W2):
  def body(r_ref, h_ref, degp_ref, b_ref, w_ref, o_ref):
    dinv = _dinv_from(degp_ref)
    z = dinv * (r_ref[0] + r_ref[1] + h_ref[...]) + b_ref[...]
    z = jnp.maximum(z, 0.0)
    o_ref[...] = dinv * jnp.dot(z, w_ref[...],
                                preferred_element_type=jnp.float32)

  grid = N // TC_BLK
  return pl.pallas_call(
      body,
      grid=(grid,),
      in_specs=[
          pl.BlockSpec((NC, TC_BLK, 32), lambda i: (0, i, 0)),
          pl.BlockSpec((TC_BLK, 32), lambda i: (i, 0)),
          pl.BlockSpec((NC, TC_BLK, 16), lambda i: (0, i, 0)),
          pl.BlockSpec((1, 32), lambda i: (0, 0)),
          pl.BlockSpec((32, 16), lambda i: (0, 0)),
      ],
      out_specs=pl.BlockSpec((TC_BLK, 16), lambda i: (i, 0)),
      out_shape=jax.ShapeDtypeStruct((N, 16), jnp.float32),
  )(r1p, h1s, degp, b1, W2)


def _tc3(r2p, h2s, degp, b2):
  def body(r_ref, h_ref, degp_ref, b_ref, o_ref):
    dinv = _dinv_from(degp_ref)
    o_ref[...] = dinv * (r_ref[0] + r_ref[1] + h_ref[...]) + b_ref[...]

  grid = N // TC_BLK
  return pl.pallas_call(
      body,
      grid=(grid,),
      in_specs=[
          pl.BlockSpec((NC, TC_BLK, 16), lambda i: (0, i, 0)),
          pl.BlockSpec((TC_BLK, 16), lambda i: (i, 0)),
          pl.BlockSpec((NC, TC_BLK, 16), lambda i: (0, i, 0)),
          pl.BlockSpec((1, 16), lambda i: (0, 0)),
      ],
      out_specs=pl.BlockSpec((TC_BLK, 16), lambda i: (i, 0)),
      out_shape=jax.ShapeDtypeStruct((N, 16), jnp.float32),
  )(r2p, h2s, degp, b2)


@jax.jit
def kernel(x, edge_index, W1, b1, W2, b2):
  ei = edge_index.astype(jnp.int32)
  pad = EP - E
  srcp = jnp.concatenate([ei[0], jnp.zeros((pad,), jnp.int32)])
  dstp = jnp.concatenate([ei[1], jnp.full((pad,), N, jnp.int32)])
  srcp = srcp.reshape(EP // BLK, BLK)
  dstp = dstp.reshape(EP // BLK, BLK)

  degp = _deg_hist(dstp)
  h1s = _tc1(x, W1, degp)
  r1p = _edge_agg(h1s, srcp, dstp, 32)
  h2s = _tc2(r1p, h1s, degp, b1.reshape(1, 32), W2)
  r2p = _edge_agg(h2s, srcp, dstp, 16)
  return _tc3(r2p, h2s, degp, b2.reshape(1, 16))


# speedup vs baseline: 29.1705x; 1.0147x over previous
import functools

import jax
import jax.numpy as jnp
from jax import lax
from jax.experimental import pallas as pl
from jax.experimental.pallas import tpu as pltpu
from jax.experimental.pallas import tpu_sc as plsc

N = 10000
E = 320000
NC, NS = 2, 16
NW = NC * NS
BLK = 128
EPT = 10240
EP = EPT * NW
NBLK = EPT // BLK
NB = 8
LA = 4
NP = 10240
ROWS_PT = NP // NS
TC_BLK = 1000


def _edge_agg(feat, srcp, dstp, F):
  mesh = plsc.VectorSubcoreMesh(core_axis_name="c", subcore_axis_name="s")

  @functools.partial(
      pl.kernel,
      out_type=jax.ShapeDtypeStruct((NC, NP, F), jnp.float32),
      mesh=mesh,
      scratch_types=[
          pltpu.VMEM((NBLK, BLK), jnp.int32),
          pltpu.VMEM((NBLK, BLK), jnp.int32),
          pltpu.VMEM((NB, BLK, F), jnp.float32),
          pltpu.VMEM((ROWS_PT, F), jnp.float32),
          pltpu.VMEM_SHARED((NP, F), jnp.float32),
          pltpu.SemaphoreType.DMA((NB,)),
          pltpu.SemaphoreType.DMA((NB,)),
      ],
      compiler_params=pltpu.CompilerParams(use_tc_tiling_on_sc=False),
  )
  def body(feat_hbm, srcp_hbm, dstp_hbm, out_hbm,
           src_v, dst_v, rows_v, zbuf_v, acc_sh, gsem, ssem):
    cid = lax.axis_index("c")
    sid = lax.axis_index("s")
    wid = sid * NC + cid

    def zero_row(r, carry):
      for j in range(F // 16):
        zbuf_v[r, pl.ds(j * 16, 16)] = jnp.zeros((16,), jnp.float32)
      return carry
    lax.fori_loop(0, ROWS_PT, zero_row, 0)
    pltpu.sync_copy(zbuf_v, acc_sh.at[pl.ds(sid * ROWS_PT, ROWS_PT)])
    plsc.subcore_barrier()

    pltpu.sync_copy(srcp_hbm.at[pl.ds(wid * NBLK, NBLK)], src_v)
    pltpu.sync_copy(dstp_hbm.at[pl.ds(wid * NBLK, NBLK)], dst_v)

    def gather(k, s):
      return pltpu.make_async_copy(
          feat_hbm.at[src_v.at[k]], rows_v.at[s], gsem.at[s])

    def scatter(k, s):
      return pltpu.make_async_copy(
          rows_v.at[s], acc_sh.at[dst_v.at[k]], ssem.at[s])

    for s in range(LA):
      gather(s, s).start()

    def group(g, carry):
      for s in range(NB):
        k = g * NB + s
        gather(k, s).wait()
        sc = scatter(k, s)
        sc.start(add=True)
        t = (s + LA) % NB

        @pl.when(k + LA < NBLK)
        def _():
          @pl.when(k >= NB - LA)
          def _():
            scatter(k + LA - NB, t).wait()
          gather(k + LA, t).start()
      return carry
    lax.fori_loop(0, NBLK // NB, group, 0)

    for k in range(NBLK - 2 * LA, NBLK):
      scatter(k, k % NB).wait()

    plsc.subcore_barrier()
    pltpu.sync_copy(acc_sh.at[pl.ds(sid * ROWS_PT, ROWS_PT)], zbuf_v)
    pltpu.sync_copy(zbuf_v, out_hbm.at[cid, pl.ds(sid * ROWS_PT, ROWS_PT)])

  return body(feat, srcp, dstp)


def _deg_hist(dstp):
  F = 16
  mesh = plsc.VectorSubcoreMesh(core_axis_name="c", subcore_axis_name="s")

  @functools.partial(
      pl.kernel,
      out_type=jax.ShapeDtypeStruct((NC, NP, F), jnp.float32),
      mesh=mesh,
      scratch_types=[
          pltpu.VMEM((NBLK, BLK), jnp.int32),
          pltpu.VMEM((BLK, F), jnp.float32),
          pltpu.VMEM((ROWS_PT, F), jnp.float32),
          pltpu.VMEM_SHARED((NP, F), jnp.float32),
      ],
      compiler_params=pltpu.CompilerParams(use_tc_tiling_on_sc=False),
  )
  def body(dstp_hbm, out_hbm, dst_v, ones_v, zbuf_v, acc_sh):
    cid = lax.axis_index("c")
    sid = lax.axis_index("s")
    wid = sid * NC + cid

    def zero_row(r, carry):
      zbuf_v[r, pl.ds(0, 16)] = jnp.zeros((16,), jnp.float32)
      return carry
    lax.fori_loop(0, ROWS_PT, zero_row, 0)
    pltpu.sync_copy(zbuf_v, acc_sh.at[pl.ds(sid * ROWS_PT, ROWS_PT)])

    def one_row(r, carry):
      ones_v[r, pl.ds(0, 16)] = jnp.ones((16,), jnp.float32)
      return carry
    lax.fori_loop(0, BLK, one_row, 0)
    plsc.subcore_barrier()

    pltpu.sync_copy(dstp_hbm.at[pl.ds(wid * NBLK, NBLK)], dst_v)

    def step(j, carry):
      pltpu.sync_copy(ones_v, acc_sh.at[dst_v.at[j]], add=True)
      return carry
    lax.fori_loop(0, NBLK, step, 0)

    plsc.subcore_barrier()
    pltpu.sync_copy(acc_sh.at[pl.ds(sid * ROWS_PT, ROWS_PT)], zbuf_v)
    pltpu.sync_copy(zbuf_v, out_hbm.at[cid, pl.ds(sid * ROWS_PT, ROWS_PT)])

  return body(dstp)


def _dinv_from(degp_blk):
  deg = degp_blk[0, :, 0:1] + degp_blk[1, :, 0:1] + 1.0
  return lax.rsqrt(deg)


def _tc1(x, W1, degp):
  def body(x_ref, w_ref, degp_ref, o_ref):
    dinv = _dinv_from(degp_ref)
    o_ref[...] = dinv * jnp.dot(x_ref[...], w_ref[...],
                                preferred_element_type=jnp.float32)

  grid = N // TC_BLK
  return pl.pallas_call(
      body,
      grid=(grid,),
      in_specs=[
          pl.BlockSpec((TC_BLK, 128), lambda i: (i, 0)),
          pl.BlockSpec((128, 32), lambda i: (0, 0)),
          pl.BlockSpec((NC, TC_BLK, 16), lambda i: (0, i, 0)),
      ],
      out_specs=pl.BlockSpec((TC_BLK, 32), lambda i: (i, 0)),
      out_shape=jax.ShapeDtypeStruct((N, 32), jnp.float32),
  )(x, W1, degp)


def _tc2(r1p, h1s, degp, b1, W2):
  def body(r_ref, h_ref, degp_ref, b_ref, w_ref, o_ref):
    dinv = _dinv_from(degp_ref)
    z = dinv * (r_ref[0] + r_ref[1] + h_ref[...]) + b_ref[...]
    z = jnp.maximum(z, 0.0)
    o_ref[...] = dinv * jnp.dot(z, w_ref[...],
                                preferred_element_type=jnp.float32)

  grid = N // TC_BLK
  return pl.pallas_call(
      body,
      grid=(grid,),
      in_specs=[
          pl.BlockSpec((NC, TC_BLK, 32), lambda i: (0, i, 0)),
          pl.BlockSpec((TC_BLK, 32), lambda i: (i, 0)),
          pl.BlockSpec((NC, TC_BLK, 16), lambda i: (0, i, 0)),
          pl.BlockSpec((1, 32), lambda i: (0, 0)),
          pl.BlockSpec((32, 16), lambda i: (0, 0)),
      ],
      out_specs=pl.BlockSpec((TC_BLK, 16), lambda i: (i, 0)),
      out_shape=jax.ShapeDtypeStruct((N, 16), jnp.float32),
  )(r1p, h1s, degp, b1, W2)


def _tc3(r2p, h2s, degp, b2):
  def body(r_ref, h_ref, degp_ref, b_ref, o_ref):
    dinv = _dinv_from(degp_ref)
    o_ref[...] = dinv * (r_ref[0] + r_ref[1] + h_ref[...]) + b_ref[...]

  grid = N // TC_BLK
  return pl.pallas_call(
      body,
      grid=(grid,),
      in_specs=[
          pl.BlockSpec((NC, TC_BLK, 16), lambda i: (0, i, 0)),
          pl.BlockSpec((TC_BLK, 16), lambda i: (i, 0)),
          pl.BlockSpec((NC, TC_BLK, 16), lambda i: (0, i, 0)),
          pl.BlockSpec((1, 16), lambda i: (0, 0)),
      ],
      out_specs=pl.BlockSpec((TC_BLK, 16), lambda i: (i, 0)),
      out_shape=jax.ShapeDtypeStruct((N, 16), jnp.float32),
  )(r2p, h2s, degp, b2)


@jax.jit
def kernel(x, edge_index, W1, b1, W2, b2):
  ei = edge_index.astype(jnp.int32)
  pad = EP - E
  srcp = jnp.concatenate([ei[0], jnp.zeros((pad,), jnp.int32)])
  dstp = jnp.concatenate([ei[1], jnp.full((pad,), N, jnp.int32)])
  srcp = srcp.reshape(EP // BLK, BLK)
  dstp = dstp.reshape(EP // BLK, BLK)

  degp = _deg_hist(dstp)
  h1s = _tc1(x, W1, degp)
  r1p = _edge_agg(h1s, srcp, dstp, 32)
  h2s = _tc2(r1p, h1s, degp, b1.reshape(1, 32), W2)
  r2p = _edge_agg(h2s, srcp, dstp, 16)
  return _tc3(r2p, h2s, degp, b2.reshape(1, 16))


# speedup vs baseline: 51.0887x; 1.7514x over previous
import functools

import jax
import jax.numpy as jnp
from jax import lax
from jax.experimental import pallas as pl
from jax.experimental.pallas import tpu as pltpu
from jax.experimental.pallas import tpu_sc as plsc

N = 10000
E = 320000
NC, NS = 2, 16
NW = NC * NS
BLK = 128
EPT = 10240
EP = EPT * NW
NBLK = EPT // BLK
NB = 8
LA = 4
NP = 10240
ROWS_PT = NP // NS
TC_BLK = 1000


def _edge_agg(feat, srcp, dstp, F):
  mesh = plsc.VectorSubcoreMesh(core_axis_name="c", subcore_axis_name="s")

  @functools.partial(
      pl.kernel,
      out_type=jax.ShapeDtypeStruct((NC, NP, F), jnp.float32),
      mesh=mesh,
      scratch_types=[
          pltpu.VMEM((NBLK, BLK), jnp.int32),
          pltpu.VMEM((NBLK, BLK), jnp.int32),
          pltpu.VMEM((NB, BLK, F), jnp.float32),
          pltpu.VMEM((ROWS_PT, F), jnp.float32),
          pltpu.VMEM_SHARED((NP, F), jnp.float32),
          pltpu.VMEM_SHARED((N, F), jnp.float32),
          pltpu.SemaphoreType.DMA((NB,)),
          pltpu.SemaphoreType.DMA((NB,)),
      ],
      compiler_params=pltpu.CompilerParams(use_tc_tiling_on_sc=False),
  )
  def body(feat_hbm, srcp_hbm, dstp_hbm, out_hbm,
           src_v, dst_v, rows_v, zbuf_v, acc_sh, feat_sh, gsem, ssem):
    cid = lax.axis_index("c")
    sid = lax.axis_index("s")
    wid = sid * NC + cid

    def zero_row(r, carry):
      for j in range(F // 16):
        zbuf_v[r, pl.ds(j * 16, 16)] = jnp.zeros((16,), jnp.float32)
      return carry
    lax.fori_loop(0, ROWS_PT, zero_row, 0)
    pltpu.sync_copy(zbuf_v, acc_sh.at[pl.ds(sid * ROWS_PT, ROWS_PT)])

    @pl.when(sid < 10)
    def _():
      pltpu.sync_copy(feat_hbm.at[pl.ds(sid * 1000, 1000)],
                      feat_sh.at[pl.ds(sid * 1000, 1000)])
    plsc.subcore_barrier()

    pltpu.sync_copy(srcp_hbm.at[pl.ds(wid * NBLK, NBLK)], src_v)
    pltpu.sync_copy(dstp_hbm.at[pl.ds(wid * NBLK, NBLK)], dst_v)

    def gather(k, s):
      return pltpu.make_async_copy(
          feat_sh.at[src_v.at[k]], rows_v.at[s], gsem.at[s])

    def scatter(k, s):
      return pltpu.make_async_copy(
          rows_v.at[s], acc_sh.at[dst_v.at[k]], ssem.at[s])

    for s in range(LA):
      gather(s, s).start()

    def group(g, carry):
      for s in range(NB):
        k = g * NB + s
        gather(k, s).wait()
        sc = scatter(k, s)
        sc.start(add=True)
        t = (s + LA) % NB

        @pl.when(k + LA < NBLK)
        def _():
          @pl.when(k >= NB - LA)
          def _():
            scatter(k + LA - NB, t).wait()
          gather(k + LA, t).start()
      return carry
    lax.fori_loop(0, NBLK // NB, group, 0)

    for k in range(NBLK - 2 * LA, NBLK):
      scatter(k, k % NB).wait()

    plsc.subcore_barrier()
    pltpu.sync_copy(acc_sh.at[pl.ds(sid * ROWS_PT, ROWS_PT)], zbuf_v)
    pltpu.sync_copy(zbuf_v, out_hbm.at[cid, pl.ds(sid * ROWS_PT, ROWS_PT)])

  return body(feat, srcp, dstp)


def _deg_hist(dstp):
  F = 16
  mesh = plsc.VectorSubcoreMesh(core_axis_name="c", subcore_axis_name="s")

  @functools.partial(
      pl.kernel,
      out_type=jax.ShapeDtypeStruct((NC, NP, F), jnp.float32),
      mesh=mesh,
      scratch_types=[
          pltpu.VMEM((NBLK, BLK), jnp.int32),
          pltpu.VMEM((BLK, F), jnp.float32),
          pltpu.VMEM((ROWS_PT, F), jnp.float32),
          pltpu.VMEM_SHARED((NP, F), jnp.float32),
      ],
      compiler_params=pltpu.CompilerParams(use_tc_tiling_on_sc=False),
  )
  def body(dstp_hbm, out_hbm, dst_v, ones_v, zbuf_v, acc_sh):
    cid = lax.axis_index("c")
    sid = lax.axis_index("s")
    wid = sid * NC + cid

    def zero_row(r, carry):
      zbuf_v[r, pl.ds(0, 16)] = jnp.zeros((16,), jnp.float32)
      return carry
    lax.fori_loop(0, ROWS_PT, zero_row, 0)
    pltpu.sync_copy(zbuf_v, acc_sh.at[pl.ds(sid * ROWS_PT, ROWS_PT)])

    def one_row(r, carry):
      ones_v[r, pl.ds(0, 16)] = jnp.ones((16,), jnp.float32)
      return carry
    lax.fori_loop(0, BLK, one_row, 0)
    plsc.subcore_barrier()

    pltpu.sync_copy(dstp_hbm.at[pl.ds(wid * NBLK, NBLK)], dst_v)

    def step(j, carry):
      pltpu.sync_copy(ones_v, acc_sh.at[dst_v.at[j]], add=True)
      return carry
    lax.fori_loop(0, NBLK, step, 0)

    plsc.subcore_barrier()
    pltpu.sync_copy(acc_sh.at[pl.ds(sid * ROWS_PT, ROWS_PT)], zbuf_v)
    pltpu.sync_copy(zbuf_v, out_hbm.at[cid, pl.ds(sid * ROWS_PT, ROWS_PT)])

  return body(dstp)


def _dinv_from(degp_blk):
  deg = degp_blk[0, :, 0:1] + degp_blk[1, :, 0:1] + 1.0
  return lax.rsqrt(deg)


def _tc1(x, W1, degp):
  def body(x_ref, w_ref, degp_ref, o_ref):
    dinv = _dinv_from(degp_ref)
    o_ref[...] = dinv * jnp.dot(x_ref[...], w_ref[...],
                                preferred_element_type=jnp.float32)

  grid = N // TC_BLK
  return pl.pallas_call(
      body,
      grid=(grid,),
      in_specs=[
          pl.BlockSpec((TC_BLK, 128), lambda i: (i, 0)),
          pl.BlockSpec((128, 32), lambda i: (0, 0)),
          pl.BlockSpec((NC, TC_BLK, 16), lambda i: (0, i, 0)),
      ],
      out_specs=pl.BlockSpec((TC_BLK, 32), lambda i: (i, 0)),
      out_shape=jax.ShapeDtypeStruct((N, 32), jnp.float32),
  )(x, W1, degp)


def _tc2(r1p, h1s, degp, b1, W2):
  def body(r_ref, h_ref, degp_ref, b_ref, w_ref, o_ref):
    dinv = _dinv_from(degp_ref)
    z = dinv * (r_ref[0] + r_ref[1] + h_ref[...]) + b_ref[...]
    z = jnp.maximum(z, 0.0)
    o_ref[...] = dinv * jnp.dot(z, w_ref[...],
                                preferred_element_type=jnp.float32)

  grid = N // TC_BLK
  return pl.pallas_call(
      body,
      grid=(grid,),
      in_specs=[
          pl.BlockSpec((NC, TC_BLK, 32), lambda i: (0, i, 0)),
          pl.BlockSpec((TC_BLK, 32), lambda i: (i, 0)),
          pl.BlockSpec((NC, TC_BLK, 16), lambda i: (0, i, 0)),
          pl.BlockSpec((1, 32), lambda i: (0, 0)),
          pl.BlockSpec((32, 16), lambda i: (0, 0)),
      ],
      out_specs=pl.BlockSpec((TC_BLK, 16), lambda i: (i, 0)),
      out_shape=jax.ShapeDtypeStruct((N, 16), jnp.float32),
  )(r1p, h1s, degp, b1, W2)


def _tc3(r2p, h2s, degp, b2):
  def body(r_ref, h_ref, degp_ref, b_ref, o_ref):
    dinv = _dinv_from(degp_ref)
    o_ref[...] = dinv * (r_ref[0] + r_ref[1] + h_ref[...]) + b_ref[...]

  grid = N // TC_BLK
  return pl.pallas_call(
      body,
      grid=(grid,),
      in_specs=[
          pl.BlockSpec((NC, TC_BLK, 16), lambda i: (0, i, 0)),
          pl.BlockSpec((TC_BLK, 16), lambda i: (i, 0)),
          pl.BlockSpec((NC, TC_BLK, 16), lambda i: (0, i, 0)),
          pl.BlockSpec((1, 16), lambda i: (0, 0)),
      ],
      out_specs=pl.BlockSpec((TC_BLK, 16), lambda i: (i, 0)),
      out_shape=jax.ShapeDtypeStruct((N, 16), jnp.float32),
  )(r2p, h2s, degp, b2)


@jax.jit
def kernel(x, edge_index, W1, b1, W2, b2):
  ei = edge_index.astype(jnp.int32)
  pad = EP - E
  srcp = jnp.concatenate([ei[0], jnp.zeros((pad,), jnp.int32)])
  dstp = jnp.concatenate([ei[1], jnp.full((pad,), N, jnp.int32)])
  srcp = srcp.reshape(EP // BLK, BLK)
  dstp = dstp.reshape(EP // BLK, BLK)

  degp = _deg_hist(dstp)
  h1s = _tc1(x, W1, degp)
  r1p = _edge_agg(h1s, srcp, dstp, 32)
  h2s = _tc2(r1p, h1s, degp, b1.reshape(1, 32), W2)
  r2p = _edge_agg(h2s, srcp, dstp, 16)
  return _tc3(r2p, h2s, degp, b2.reshape(1, 16))


# speedup vs baseline: 53.3926x; 1.0451x over previous
import functools

import jax
import jax.numpy as jnp
from jax import lax
from jax.experimental import pallas as pl
from jax.experimental.pallas import tpu as pltpu
from jax.experimental.pallas import tpu_sc as plsc

N = 10000
E = 320000
NC, NS = 2, 16
NW = NC * NS
BLK = 128
TBLK = E // BLK
NBLK = TBLK // NW
XTRA = TBLK % NW
NB = 6
LA = 3
NP = 10240
ROWS_PT = NP // NS
TC_BLK = 1000


def _edge_agg(feat, srcp, dstp, F):
  mesh = plsc.VectorSubcoreMesh(core_axis_name="c", subcore_axis_name="s")

  @functools.partial(
      pl.kernel,
      out_type=jax.ShapeDtypeStruct((NC, NP, F), jnp.float32),
      mesh=mesh,
      scratch_types=[
          pltpu.VMEM((NBLK + 1, BLK), jnp.int32),
          pltpu.VMEM((NBLK + 1, BLK), jnp.int32),
          pltpu.VMEM((NB, BLK, F), jnp.float32),
          pltpu.VMEM((ROWS_PT, F), jnp.float32),
          pltpu.VMEM_SHARED((NP, F), jnp.float32),
          pltpu.VMEM_SHARED((N, F), jnp.float32),
          pltpu.SemaphoreType.DMA((NB,)),
          pltpu.SemaphoreType.DMA((NB,)),
      ],
      compiler_params=pltpu.CompilerParams(use_tc_tiling_on_sc=False),
  )
  def body(feat_hbm, srcp_hbm, dstp_hbm, out_hbm,
           src_v, dst_v, rows_v, zbuf_v, acc_sh, feat_sh, gsem, ssem):
    cid = lax.axis_index("c")
    sid = lax.axis_index("s")
    wid = sid * NC + cid

    def zero_row(r, carry):
      for j in range(F // 16):
        zbuf_v[r, pl.ds(j * 16, 16)] = jnp.zeros((16,), jnp.float32)
      return carry
    lax.fori_loop(0, ROWS_PT, zero_row, 0)
    pltpu.sync_copy(zbuf_v, acc_sh.at[pl.ds(sid * ROWS_PT, ROWS_PT)])

    @pl.when(sid < 10)
    def _():
      pltpu.sync_copy(feat_hbm.at[pl.ds(sid * 1000, 1000)],
                      feat_sh.at[pl.ds(sid * 1000, 1000)])
    plsc.subcore_barrier()

    base = wid * NBLK + jnp.minimum(wid, XTRA)
    pltpu.sync_copy(srcp_hbm.at[pl.ds(base, NBLK)], src_v.at[pl.ds(0, NBLK)])
    pltpu.sync_copy(dstp_hbm.at[pl.ds(base, NBLK)], dst_v.at[pl.ds(0, NBLK)])

    @pl.when(wid < XTRA)
    def _():
      pltpu.sync_copy(srcp_hbm.at[pl.ds(base + NBLK, 1)],
                      src_v.at[pl.ds(NBLK, 1)])
      pltpu.sync_copy(dstp_hbm.at[pl.ds(base + NBLK, 1)],
                      dst_v.at[pl.ds(NBLK, 1)])

    def gather(k, s):
      return pltpu.make_async_copy(
          feat_sh.at[src_v.at[k]], rows_v.at[s], gsem.at[s])

    def scatter(k, s):
      return pltpu.make_async_copy(
          rows_v.at[s], acc_sh.at[dst_v.at[k]], ssem.at[s])

    for s in range(LA):
      gather(s, s).start()

    def group(g, carry):
      for s in range(NB):
        k = g * NB + s
        gather(k, s).wait()
        sc = scatter(k, s)
        sc.start(add=True)
        t = (s + LA) % NB

        @pl.when(k + LA < NBLK)
        def _():
          @pl.when(k >= NB - LA)
          def _():
            scatter(k + LA - NB, t).wait()
          gather(k + LA, t).start()
      return carry
    lax.fori_loop(0, NBLK // NB, group, 0)

    for k in range(NBLK - 2 * LA, NBLK):
      scatter(k, k % NB).wait()

    @pl.when(wid < XTRA)
    def _():
      gather(NBLK, 0).start()
      gather(NBLK, 0).wait()
      sc = scatter(NBLK, 0)
      sc.start(add=True)
      sc.wait()

    plsc.subcore_barrier()
    pltpu.sync_copy(acc_sh.at[pl.ds(sid * ROWS_PT, ROWS_PT)], zbuf_v)
    pltpu.sync_copy(zbuf_v, out_hbm.at[cid, pl.ds(sid * ROWS_PT, ROWS_PT)])

  return body(feat, srcp, dstp)


def _deg_hist(dstp):
  F = 16
  mesh = plsc.VectorSubcoreMesh(core_axis_name="c", subcore_axis_name="s")

  @functools.partial(
      pl.kernel,
      out_type=jax.ShapeDtypeStruct((NC, NP, F), jnp.float32),
      mesh=mesh,
      scratch_types=[
          pltpu.VMEM((NBLK + 1, BLK), jnp.int32),
          pltpu.VMEM((BLK, F), jnp.float32),
          pltpu.VMEM((ROWS_PT, F), jnp.float32),
          pltpu.VMEM_SHARED((NP, F), jnp.float32),
      ],
      compiler_params=pltpu.CompilerParams(use_tc_tiling_on_sc=False),
  )
  def body(dstp_hbm, out_hbm, dst_v, ones_v, zbuf_v, acc_sh):
    cid = lax.axis_index("c")
    sid = lax.axis_index("s")
    wid = sid * NC + cid

    def zero_row(r, carry):
      zbuf_v[r, pl.ds(0, 16)] = jnp.zeros((16,), jnp.float32)
      return carry
    lax.fori_loop(0, ROWS_PT, zero_row, 0)
    pltpu.sync_copy(zbuf_v, acc_sh.at[pl.ds(sid * ROWS_PT, ROWS_PT)])

    def one_row(r, carry):
      ones_v[r, pl.ds(0, 16)] = jnp.ones((16,), jnp.float32)
      return carry
    lax.fori_loop(0, BLK, one_row, 0)
    plsc.subcore_barrier()

    base = wid * NBLK + jnp.minimum(wid, XTRA)
    pltpu.sync_copy(dstp_hbm.at[pl.ds(base, NBLK)], dst_v.at[pl.ds(0, NBLK)])

    @pl.when(wid < XTRA)
    def _():
      pltpu.sync_copy(dstp_hbm.at[pl.ds(base + NBLK, 1)],
                      dst_v.at[pl.ds(NBLK, 1)])

    nblk = NBLK + jnp.where(wid < XTRA, 1, 0)

    def step(j, carry):
      pltpu.sync_copy(ones_v, acc_sh.at[dst_v.at[j]], add=True)
      return carry
    lax.fori_loop(0, nblk, step, 0)

    plsc.subcore_barrier()
    pltpu.sync_copy(acc_sh.at[pl.ds(sid * ROWS_PT, ROWS_PT)], zbuf_v)
    pltpu.sync_copy(zbuf_v, out_hbm.at[cid, pl.ds(sid * ROWS_PT, ROWS_PT)])

  return body(dstp)


def _dinv_from(degp_blk):
  deg = degp_blk[0, :, 0:1] + degp_blk[1, :, 0:1] + 1.0
  return lax.rsqrt(deg)


def _tc1(x, W1, degp):
  def body(x_ref, w_ref, degp_ref, o_ref):
    dinv = _dinv_from(degp_ref)
    o_ref[...] = dinv * jnp.dot(x_ref[...], w_ref[...],
                                preferred_element_type=jnp.float32)

  grid = N // TC_BLK
  return pl.pallas_call(
      body,
      grid=(grid,),
      in_specs=[
          pl.BlockSpec((TC_BLK, 128), lambda i: (i, 0)),
          pl.BlockSpec((128, 32), lambda i: (0, 0)),
          pl.BlockSpec((NC, TC_BLK, 16), lambda i: (0, i, 0)),
      ],
      out_specs=pl.BlockSpec((TC_BLK, 32), lambda i: (i, 0)),
      out_shape=jax.ShapeDtypeStruct((N, 32), jnp.float32),
  )(x, W1, degp)


def _tc2(r1p, h1s, degp, b1, W2):
  def body(r_ref, h_ref, degp_ref, b_ref, w_ref, o_ref):
    dinv = _dinv_from(degp_ref)
    z = dinv * (r_ref[0] + r_ref[1] + h_ref[...]) + b_ref[...]
    z = jnp.maximum(z, 0.0)
    o_ref[...] = dinv * jnp.dot(z, w_ref[...],
                                preferred_element_type=jnp.float32)

  grid = N // TC_BLK
  return pl.pallas_call(
      body,
      grid=(grid,),
      in_specs=[
          pl.BlockSpec((NC, TC_BLK, 32), lambda i: (0, i, 0)),
          pl.BlockSpec((TC_BLK, 32), lambda i: (i, 0)),
          pl.BlockSpec((NC, TC_BLK, 16), lambda i: (0, i, 0)),
          pl.BlockSpec((1, 32), lambda i: (0, 0)),
          pl.BlockSpec((32, 16), lambda i: (0, 0)),
      ],
      out_specs=pl.BlockSpec((TC_BLK, 16), lambda i: (i, 0)),
      out_shape=jax.ShapeDtypeStruct((N, 16), jnp.float32),
  )(r1p, h1s, degp, b1, W2)


def _tc3(r2p, h2s, degp, b2):
  def body(r_ref, h_ref, degp_ref, b_ref, o_ref):
    dinv = _dinv_from(degp_ref)
    o_ref[...] = dinv * (r_ref[0] + r_ref[1] + h_ref[...]) + b_ref[...]

  grid = N // TC_BLK
  return pl.pallas_call(
      body,
      grid=(grid,),
      in_specs=[
          pl.BlockSpec((NC, TC_BLK, 16), lambda i: (0, i, 0)),
          pl.BlockSpec((TC_BLK, 16), lambda i: (i, 0)),
          pl.BlockSpec((NC, TC_BLK, 16), lambda i: (0, i, 0)),
          pl.BlockSpec((1, 16), lambda i: (0, 0)),
      ],
      out_specs=pl.BlockSpec((TC_BLK, 16), lambda i: (i, 0)),
      out_shape=jax.ShapeDtypeStruct((N, 16), jnp.float32),
  )(r2p, h2s, degp, b2)


@jax.jit
def kernel(x, edge_index, W1, b1, W2, b2):
  ei = edge_index.astype(jnp.int32)
  srcp = ei[0].reshape(TBLK, BLK)
  dstp = ei[1].reshape(TBLK, BLK)

  degp = _deg_hist(dstp)
  h1s = _tc1(x, W1, degp)
  r1p = _edge_agg(h1s, srcp, dstp, 32)
  h2s = _tc2(r1p, h1s, degp, b1.reshape(1, 32), W2)
  r2p = _edge_agg(h2s, srcp, dstp, 16)
  return _tc3(r2p, h2s, degp, b2.reshape(1, 16))


# speedup vs baseline: 55.3281x; 1.0362x over previous
import functools

import jax
import jax.numpy as jnp
from jax import lax
from jax.experimental import pallas as pl
from jax.experimental.pallas import tpu as pltpu
from jax.experimental.pallas import tpu_sc as plsc

N = 10000
E = 320000
NC, NS = 2, 16
NW = NC * NS
BLK = 128
TBLK = E // BLK
NBLK = TBLK // NW
XTRA = TBLK % NW
NB = 6
LA = 3
NP = 10240
ROWS_PT = NP // NS
TC_BLK = 5000


def _edge_agg(feat, srcp, dstp, F):
  mesh = plsc.VectorSubcoreMesh(core_axis_name="c", subcore_axis_name="s")

  @functools.partial(
      pl.kernel,
      out_type=jax.ShapeDtypeStruct((NC, NP, F), jnp.float32),
      mesh=mesh,
      scratch_types=[
          pltpu.VMEM((NBLK + 1, BLK), jnp.int32),
          pltpu.VMEM((NBLK + 1, BLK), jnp.int32),
          pltpu.VMEM((NB, BLK, F), jnp.float32),
          pltpu.VMEM((ROWS_PT, F), jnp.float32),
          pltpu.VMEM_SHARED((NP, F), jnp.float32),
          pltpu.VMEM_SHARED((N, F), jnp.float32),
          pltpu.SemaphoreType.DMA((NB,)),
          pltpu.SemaphoreType.DMA((NB,)),
      ],
      compiler_params=pltpu.CompilerParams(use_tc_tiling_on_sc=False),
  )
  def body(feat_hbm, srcp_hbm, dstp_hbm, out_hbm,
           src_v, dst_v, rows_v, zbuf_v, acc_sh, feat_sh, gsem, ssem):
    cid = lax.axis_index("c")
    sid = lax.axis_index("s")
    wid = sid * NC + cid

    def zero_row(r, carry):
      for j in range(F // 16):
        zbuf_v[r, pl.ds(j * 16, 16)] = jnp.zeros((16,), jnp.float32)
      return carry
    lax.fori_loop(0, ROWS_PT, zero_row, 0)
    pltpu.sync_copy(zbuf_v, acc_sh.at[pl.ds(sid * ROWS_PT, ROWS_PT)])

    @pl.when(sid < 10)
    def _():
      pltpu.sync_copy(feat_hbm.at[pl.ds(sid * 1000, 1000)],
                      feat_sh.at[pl.ds(sid * 1000, 1000)])
    plsc.subcore_barrier()

    base = wid * NBLK + jnp.minimum(wid, XTRA)
    pltpu.sync_copy(srcp_hbm.at[pl.ds(base, NBLK)], src_v.at[pl.ds(0, NBLK)])
    pltpu.sync_copy(dstp_hbm.at[pl.ds(base, NBLK)], dst_v.at[pl.ds(0, NBLK)])

    @pl.when(wid < XTRA)
    def _():
      pltpu.sync_copy(srcp_hbm.at[pl.ds(base + NBLK, 1)],
                      src_v.at[pl.ds(NBLK, 1)])
      pltpu.sync_copy(dstp_hbm.at[pl.ds(base + NBLK, 1)],
                      dst_v.at[pl.ds(NBLK, 1)])

    def gather(k, s):
      return pltpu.make_async_copy(
          feat_sh.at[src_v.at[k]], rows_v.at[s], gsem.at[s])

    def scatter(k, s):
      return pltpu.make_async_copy(
          rows_v.at[s], acc_sh.at[dst_v.at[k]], ssem.at[s])

    for s in range(LA):
      gather(s, s).start()

    def group(g, carry):
      for s in range(NB):
        k = g * NB + s
        gather(k, s).wait()
        sc = scatter(k, s)
        sc.start(add=True)
        t = (s + LA) % NB

        @pl.when(k + LA < NBLK)
        def _():
          @pl.when(k >= NB - LA)
          def _():
            scatter(k + LA - NB, t).wait()
          gather(k + LA, t).start()
      return carry
    lax.fori_loop(0, NBLK // NB, group, 0)

    for k in range(NBLK - 2 * LA, NBLK):
      scatter(k, k % NB).wait()

    @pl.when(wid < XTRA)
    def _():
      gather(NBLK, 0).start()
      gather(NBLK, 0).wait()
      sc = scatter(NBLK, 0)
      sc.start(add=True)
      sc.wait()

    plsc.subcore_barrier()
    pltpu.sync_copy(acc_sh.at[pl.ds(sid * ROWS_PT, ROWS_PT)], zbuf_v)
    pltpu.sync_copy(zbuf_v, out_hbm.at[cid, pl.ds(sid * ROWS_PT, ROWS_PT)])

  return body(feat, srcp, dstp)


def _deg_hist(dstp):
  F = 16
  mesh = plsc.VectorSubcoreMesh(core_axis_name="c", subcore_axis_name="s")

  @functools.partial(
      pl.kernel,
      out_type=jax.ShapeDtypeStruct((NC, NP, F), jnp.float32),
      mesh=mesh,
      scratch_types=[
          pltpu.VMEM((NBLK + 1, BLK), jnp.int32),
          pltpu.VMEM((BLK, F), jnp.float32),
          pltpu.VMEM((ROWS_PT, F), jnp.float32),
          pltpu.VMEM_SHARED((NP, F), jnp.float32),
      ],
      compiler_params=pltpu.CompilerParams(use_tc_tiling_on_sc=False),
  )
  def body(dstp_hbm, out_hbm, dst_v, ones_v, zbuf_v, acc_sh):
    cid = lax.axis_index("c")
    sid = lax.axis_index("s")
    wid = sid * NC + cid

    def zero_row(r, carry):
      zbuf_v[r, pl.ds(0, 16)] = jnp.zeros((16,), jnp.float32)
      return carry
    lax.fori_loop(0, ROWS_PT, zero_row, 0)
    pltpu.sync_copy(zbuf_v, acc_sh.at[pl.ds(sid * ROWS_PT, ROWS_PT)])

    def one_row(r, carry):
      ones_v[r, pl.ds(0, 16)] = jnp.ones((16,), jnp.float32)
      return carry
    lax.fori_loop(0, BLK, one_row, 0)
    plsc.subcore_barrier()

    base = wid * NBLK + jnp.minimum(wid, XTRA)
    pltpu.sync_copy(dstp_hbm.at[pl.ds(base, NBLK)], dst_v.at[pl.ds(0, NBLK)])

    @pl.when(wid < XTRA)
    def _():
      pltpu.sync_copy(dstp_hbm.at[pl.ds(base + NBLK, 1)],
                      dst_v.at[pl.ds(NBLK, 1)])

    nblk = NBLK + jnp.where(wid < XTRA, 1, 0)

    def step(j, carry):
      pltpu.sync_copy(ones_v, acc_sh.at[dst_v.at[j]], add=True)
      return carry
    lax.fori_loop(0, nblk, step, 0)

    plsc.subcore_barrier()
    pltpu.sync_copy(acc_sh.at[pl.ds(sid * ROWS_PT, ROWS_PT)], zbuf_v)
    pltpu.sync_copy(zbuf_v, out_hbm.at[cid, pl.ds(sid * ROWS_PT, ROWS_PT)])

  return body(dstp)


def _dinv_from(degp_blk):
  deg = degp_blk[0, :, 0:1] + degp_blk[1, :, 0:1] + 1.0
  return lax.rsqrt(deg)


def _tc1(x, W1, degp):
  def body(x_ref, w_ref, degp_ref, o_ref):
    dinv = _dinv_from(degp_ref)
    o_ref[...] = dinv * jnp.dot(x_ref[...], w_ref[...],
                                preferred_element_type=jnp.float32)

  grid = N // TC_BLK
  return pl.pallas_call(
      body,
      grid=(grid,),
      in_specs=[
          pl.BlockSpec((TC_BLK, 128), lambda i: (i, 0)),
          pl.BlockSpec((128, 32), lambda i: (0, 0)),
          pl.BlockSpec((NC, TC_BLK, 16), lambda i: (0, i, 0)),
      ],
      out_specs=pl.BlockSpec((TC_BLK, 32), lambda i: (i, 0)),
      out_shape=jax.ShapeDtypeStruct((N, 32), jnp.float32),
  )(x, W1, degp)


def _tc2(r1p, h1s, degp, b1, W2):
  def body(r_ref, h_ref, degp_ref, b_ref, w_ref, o_ref):
    dinv = _dinv_from(degp_ref)
    z = dinv * (r_ref[0] + r_ref[1] + h_ref[...]) + b_ref[...]
    z = jnp.maximum(z, 0.0)
    o_ref[...] = dinv * jnp.dot(z, w_ref[...],
                                preferred_element_type=jnp.float32)

  grid = N // TC_BLK
  return pl.pallas_call(
      body,
      grid=(grid,),
      in_specs=[
          pl.BlockSpec((NC, TC_BLK, 32), lambda i: (0, i, 0)),
          pl.BlockSpec((TC_BLK, 32), lambda i: (i, 0)),
          pl.BlockSpec((NC, TC_BLK, 16), lambda i: (0, i, 0)),
          pl.BlockSpec((1, 32), lambda i: (0, 0)),
          pl.BlockSpec((32, 16), lambda i: (0, 0)),
      ],
      out_specs=pl.BlockSpec((TC_BLK, 16), lambda i: (i, 0)),
      out_shape=jax.ShapeDtypeStruct((N, 16), jnp.float32),
  )(r1p, h1s, degp, b1, W2)


def _tc3(r2p, h2s, degp, b2):
  def body(r_ref, h_ref, degp_ref, b_ref, o_ref):
    dinv = _dinv_from(degp_ref)
    o_ref[...] = dinv * (r_ref[0] + r_ref[1] + h_ref[...]) + b_ref[...]

  grid = N // TC_BLK
  return pl.pallas_call(
      body,
      grid=(grid,),
      in_specs=[
          pl.BlockSpec((NC, TC_BLK, 16), lambda i: (0, i, 0)),
          pl.BlockSpec((TC_BLK, 16), lambda i: (i, 0)),
          pl.BlockSpec((NC, TC_BLK, 16), lambda i: (0, i, 0)),
          pl.BlockSpec((1, 16), lambda i: (0, 0)),
      ],
      out_specs=pl.BlockSpec((TC_BLK, 16), lambda i: (i, 0)),
      out_shape=jax.ShapeDtypeStruct((N, 16), jnp.float32),
  )(r2p, h2s, degp, b2)


@jax.jit
def kernel(x, edge_index, W1, b1, W2, b2):
  ei = edge_index.astype(jnp.int32)
  srcp = ei[0].reshape(TBLK, BLK)
  dstp = ei[1].reshape(TBLK, BLK)

  degp = _deg_hist(dstp)
  h1s = _tc1(x, W1, degp)
  r1p = _edge_agg(h1s, srcp, dstp, 32)
  h2s = _tc2(r1p, h1s, degp, b1.reshape(1, 32), W2)
  r2p = _edge_agg(h2s, srcp, dstp, 16)
  return _tc3(r2p, h2s, degp, b2.reshape(1, 16))
